# Initial kernel scaffold; baseline (speedup 1.0000x reference)
import jax
import jax.numpy as jnp
from jax import lax
from jax.experimental import pallas as pl
from jax.experimental.pallas import tpu as pltpu
from jax.experimental.pallas import tpu_sc as plsc

_T = 1000
_B = 16384
_NC = 2
_NS = 16
_L = 16
_NW = _NC * _NS
_BPW = _B // _NW
_CHUNKS = _BPW // _L


def _sc_body(steps_hbm, a_hbm, ab_hbm, abp_hbm, out_hbm,
             idx_v, ta_v, tab_v, tabp_v, oa_v, oab_v, oabp_v):
    wid = lax.axis_index("s") * _NC + lax.axis_index("c")
    base = wid * _BPW
    pltpu.sync_copy(steps_hbm.at[pl.ds(base, _BPW)], idx_v)
    pltpu.sync_copy(a_hbm, ta_v)
    pltpu.sync_copy(ab_hbm, tab_v)
    pltpu.sync_copy(abp_hbm, tabp_v)

    def chunk(i, carry):
        off = i * _L
        iv = idx_v[pl.ds(off, _L)]
        oa_v[pl.ds(off, _L)] = plsc.load_gather(ta_v, [iv])
        oab_v[pl.ds(off, _L)] = plsc.load_gather(tab_v, [iv])
        oabp_v[pl.ds(off, _L)] = plsc.load_gather(tabp_v, [iv])
        return carry

    lax.fori_loop(0, _CHUNKS, chunk, 0)

    pltpu.sync_copy(oa_v, out_hbm.at[0, pl.ds(base, _BPW)])
    pltpu.sync_copy(oab_v, out_hbm.at[1, pl.ds(base, _BPW)])
    pltpu.sync_copy(oabp_v, out_hbm.at[2, pl.ds(base, _BPW)])


def kernel(diffusion_steps, alphas, alpha_bars, alpha_bars_prev):
    mesh = plsc.VectorSubcoreMesh(core_axis_name="c", subcore_axis_name="s")
    out = pl.kernel(
        _sc_body,
        out_type=jax.ShapeDtypeStruct((3, _B), jnp.float32),
        mesh=mesh,
        scratch_types=[
            pltpu.VMEM((_BPW,), jnp.int32),
            pltpu.VMEM((_T,), jnp.float32),
            pltpu.VMEM((_T,), jnp.float32),
            pltpu.VMEM((_T,), jnp.float32),
            pltpu.VMEM((_BPW,), jnp.float32),
            pltpu.VMEM((_BPW,), jnp.float32),
            pltpu.VMEM((_BPW,), jnp.float32),
        ],
    )(diffusion_steps, alphas, alpha_bars, alpha_bars_prev)
    return out.reshape(3, _B, 1, 1, 1)


# speedup vs baseline: 12.5041x; 12.5041x over previous
import jax
import jax.numpy as jnp
from jax import lax
from jax.experimental import pallas as pl
from jax.experimental.pallas import tpu as pltpu
from jax.experimental.pallas import tpu_sc as plsc

_T = 1000
_B = 16384
_NC = 2
_NS = 16
_L = 16
_NW = _NC * _NS
_BPW = _B // _NW
_CHUNKS = _BPW // _L


def _sc_body(steps_hbm, a_hbm, ab_hbm, abp_hbm, out_hbm,
             idx_v, ta_v, tab_v, tabp_v, oa_v, oab_v, oabp_v):
    wid = lax.axis_index("s") * _NC + lax.axis_index("c")
    base = wid * _BPW
    pltpu.sync_copy(steps_hbm.at[pl.ds(base, _BPW)], idx_v)
    pltpu.sync_copy(a_hbm, ta_v)
    pltpu.sync_copy(ab_hbm, tab_v)
    pltpu.sync_copy(abp_hbm, tabp_v)

    def chunk(i, carry):
        off = i * _L
        iv = idx_v[pl.ds(off, _L)]
        oa_v[pl.ds(off, _L)] = plsc.load_gather(ta_v, [iv])
        oab_v[pl.ds(off, _L)] = plsc.load_gather(tab_v, [iv])
        oabp_v[pl.ds(off, _L)] = plsc.load_gather(tabp_v, [iv])
        return carry

    lax.fori_loop(0, _CHUNKS, chunk, 0)

    pltpu.sync_copy(oa_v, out_hbm.at[pl.ds(base, _BPW)])
    pltpu.sync_copy(oab_v, out_hbm.at[pl.ds(_B + base, _BPW)])
    pltpu.sync_copy(oabp_v, out_hbm.at[pl.ds(2 * _B + base, _BPW)])


def kernel(diffusion_steps, alphas, alpha_bars, alpha_bars_prev):
    mesh = plsc.VectorSubcoreMesh(core_axis_name="c", subcore_axis_name="s")
    out = pl.kernel(
        _sc_body,
        out_type=jax.ShapeDtypeStruct((3 * _B,), jnp.float32),
        mesh=mesh,
        compiler_params=pltpu.CompilerParams(needs_layout_passes=False),
        scratch_types=[
            pltpu.VMEM((_BPW,), jnp.int32),
            pltpu.VMEM((_T,), jnp.float32),
            pltpu.VMEM((_T,), jnp.float32),
            pltpu.VMEM((_T,), jnp.float32),
            pltpu.VMEM((_BPW,), jnp.float32),
            pltpu.VMEM((_BPW,), jnp.float32),
            pltpu.VMEM((_BPW,), jnp.float32),
        ],
    )(diffusion_steps, alphas, alpha_bars, alpha_bars_prev)
    return out.reshape(3, _B, 1, 1, 1)


# speedup vs baseline: 13.4014x; 1.0718x over previous
import jax
import jax.numpy as jnp
from jax import lax
from jax.experimental import pallas as pl
from jax.experimental.pallas import tpu as pltpu
from jax.experimental.pallas import tpu_sc as plsc

_T = 1000
_B = 16384
_NC = 2
_NS = 16
_L = 16
_NW = _NC * _NS
_BPW = _B // _NW
_CHUNKS = _BPW // _L


def _sc_body(steps_hbm, a_hbm, ab_hbm, abp_hbm, out_hbm,
             idx_v, ta_v, tab_v, tabp_v, oa_v, oab_v, oabp_v, sem):
    wid = lax.axis_index("s") * _NC + lax.axis_index("c")
    base = wid * _BPW
    c0 = pltpu.async_copy(steps_hbm.at[pl.ds(base, _BPW)], idx_v, sem)
    c1 = pltpu.async_copy(a_hbm, ta_v, sem)
    c2 = pltpu.async_copy(ab_hbm, tab_v, sem)
    c3 = pltpu.async_copy(abp_hbm, tabp_v, sem)
    c0.wait()
    c1.wait()
    c2.wait()
    c3.wait()

    @plsc.parallel_loop(0, _BPW, step=_L, unroll=4)
    def _chunk(off):
        iv = idx_v[pl.ds(off, _L)]
        oa_v[pl.ds(off, _L)] = plsc.load_gather(ta_v, [iv])
        oab_v[pl.ds(off, _L)] = plsc.load_gather(tab_v, [iv])
        oabp_v[pl.ds(off, _L)] = plsc.load_gather(tabp_v, [iv])

    s0 = pltpu.async_copy(oa_v, out_hbm.at[pl.ds(base, _BPW)], sem)
    s1 = pltpu.async_copy(oab_v, out_hbm.at[pl.ds(_B + base, _BPW)], sem)
    s2 = pltpu.async_copy(oabp_v, out_hbm.at[pl.ds(2 * _B + base, _BPW)], sem)
    s0.wait()
    s1.wait()
    s2.wait()


def kernel(diffusion_steps, alphas, alpha_bars, alpha_bars_prev):
    mesh = plsc.VectorSubcoreMesh(core_axis_name="c", subcore_axis_name="s")
    out = pl.kernel(
        _sc_body,
        out_type=jax.ShapeDtypeStruct((3 * _B,), jnp.float32),
        mesh=mesh,
        compiler_params=pltpu.CompilerParams(
            needs_layout_passes=False, skip_device_barrier=True),
        scratch_types=[
            pltpu.VMEM((_BPW,), jnp.int32),
            pltpu.VMEM((_T,), jnp.float32),
            pltpu.VMEM((_T,), jnp.float32),
            pltpu.VMEM((_T,), jnp.float32),
            pltpu.VMEM((_BPW,), jnp.float32),
            pltpu.VMEM((_BPW,), jnp.float32),
            pltpu.VMEM((_BPW,), jnp.float32),
            pltpu.SemaphoreType.DMA,
        ],
    )(diffusion_steps, alphas, alpha_bars, alpha_bars_prev)
    return out.reshape(3, _B, 1, 1, 1)


# speedup vs baseline: 14.8794x; 1.1103x over previous
import jax
import jax.numpy as jnp
from jax import lax
from jax.experimental import pallas as pl
from jax.experimental.pallas import tpu as pltpu
from jax.experimental.pallas import tpu_sc as plsc

_T = 1000
_B = 16384
_NC = 1
_NS = 16
_L = 16
_NW = _NC * _NS
_BPW = _B // _NW
_CHUNKS = _BPW // _L


def _sc_body(steps_hbm, a_hbm, ab_hbm, abp_hbm, out_hbm,
             idx_v, ta_v, tab_v, tabp_v, oa_v, oab_v, oabp_v, sem):
    wid = lax.axis_index("s") * _NC + lax.axis_index("c")
    base = wid * _BPW
    c0 = pltpu.async_copy(steps_hbm.at[pl.ds(base, _BPW)], idx_v, sem)
    c1 = pltpu.async_copy(a_hbm, ta_v, sem)
    c2 = pltpu.async_copy(ab_hbm, tab_v, sem)
    c3 = pltpu.async_copy(abp_hbm, tabp_v, sem)
    c0.wait()
    c1.wait()
    c2.wait()
    c3.wait()

    @plsc.parallel_loop(0, _BPW, step=_L, unroll=4)
    def _chunk(off):
        iv = idx_v[pl.ds(off, _L)]
        oa_v[pl.ds(off, _L)] = plsc.load_gather(ta_v, [iv])
        oab_v[pl.ds(off, _L)] = plsc.load_gather(tab_v, [iv])
        oabp_v[pl.ds(off, _L)] = plsc.load_gather(tabp_v, [iv])

    s0 = pltpu.async_copy(oa_v, out_hbm.at[pl.ds(base, _BPW)], sem)
    s1 = pltpu.async_copy(oab_v, out_hbm.at[pl.ds(_B + base, _BPW)], sem)
    s2 = pltpu.async_copy(oabp_v, out_hbm.at[pl.ds(2 * _B + base, _BPW)], sem)
    s0.wait()
    s1.wait()
    s2.wait()


def kernel(diffusion_steps, alphas, alpha_bars, alpha_bars_prev):
    mesh = plsc.VectorSubcoreMesh(core_axis_name="c", subcore_axis_name="s", num_cores=1)
    out = pl.kernel(
        _sc_body,
        out_type=jax.ShapeDtypeStruct((3 * _B,), jnp.float32),
        mesh=mesh,
        compiler_params=pltpu.CompilerParams(
            needs_layout_passes=False, skip_device_barrier=True),
        scratch_types=[
            pltpu.VMEM((_BPW,), jnp.int32),
            pltpu.VMEM((_T,), jnp.float32),
            pltpu.VMEM((_T,), jnp.float32),
            pltpu.VMEM((_T,), jnp.float32),
            pltpu.VMEM((_BPW,), jnp.float32),
            pltpu.VMEM((_BPW,), jnp.float32),
            pltpu.VMEM((_BPW,), jnp.float32),
            pltpu.SemaphoreType.DMA,
        ],
    )(diffusion_steps, alphas, alpha_bars, alpha_bars_prev)
    return out.reshape(3, _B, 1, 1, 1)
